# baseline (speedup 1.0000x reference)
import jax
import jax.numpy as jnp
from jax import lax
from jax.experimental import pallas as pl
from jax.experimental.pallas import tpu as pltpu
from jax.experimental.pallas import tpu_sc as plsc

N = 10000
E = 320000
D = 128

NT = 16
LANES = 16
CH = 128
NR = 80
NPAD = NR * CH
RPT = NR // NT
SLICE = RPT * CH
VPR = CH // LANES

ECHUNK = 19968
EMAX = E - 15 * ECHUNK
NLOOP = EMAX // LANES
PADNODE = NPAD - 1


def _rsqrt_newton(x):
    i = lax.bitcast_convert_type(x, jnp.int32)
    y = lax.bitcast_convert_type(jnp.int32(0x5F3759DF) - (i >> 1), jnp.float32)
    for _ in range(3):
        y = y * (1.5 - 0.5 * x * y * y)
    return y


def _zero_acc(acc):
    z = jnp.zeros((LANES,), jnp.float32)

    def body(r, _):
        for k in range(VPR):
            acc[r, pl.ds(k * LANES, LANES)] = z
        return 0

    lax.fori_loop(0, NR, body, 0)


def _sc_body(edge_hbm, h0_hbm, consts_hbm, out_hbm,
             se2, pk, cfull, acc_a, acc_b, acc_c, row_idx, cvec,
             invo_s, invi_s, work_s, c_s, sums3, tmp_all, stv, sem0,
             sh_a, sh_b, sh_c, sh_sums):
    sid = lax.axis_index("s")
    cid = lax.axis_index("c")

    with jax.named_scope("p0_stage"):
        cp = pltpu.async_copy(edge_hbm.at[:, pl.ds(sid * ECHUNK, ECHUNK)],
                              se2.at[:, pl.ds(0, ECHUNK)], sem0)
        pltpu.sync_copy(consts_hbm, cvec)
        for k in range(NR // LANES):
            row_idx[pl.ds(k * LANES, LANES)] = lax.iota(jnp.int32, LANES) + (k * LANES)
        _zero_acc(acc_a)
        _zero_acc(acc_b)
        _zero_acc(acc_c)
        cp.wait()

        @pl.when(sid < 15)
        def _():
            padv = jnp.full((LANES,), PADNODE, jnp.int32)
            for k in range((EMAX - ECHUNK) // LANES):
                se2[0, pl.ds(ECHUNK + k * LANES, LANES)] = padv
                se2[1, pl.ds(ECHUNK + k * LANES, LANES)] = padv

        @pl.when(sid == 15)
        def _():
            pltpu.sync_copy(edge_hbm.at[:, pl.ds(E - (EMAX - ECHUNK), EMAX - ECHUNK)],
                            se2.at[:, pl.ds(ECHUNK, EMAX - ECHUNK)])

    @pl.when(sid == 0)
    def _():
        pltpu.sync_copy(acc_a, sh_a)
        pltpu.sync_copy(acc_b, sh_b)

    plsc.subcore_barrier()

    ones = jnp.ones((LANES,), jnp.float32)

    with jax.named_scope("deg_loop"):
        @plsc.parallel_loop(0, NLOOP, unroll=8)
        def deg_body(e):
            sl = pl.ds(e * LANES, LANES)
            s = se2[0, sl]
            d = se2[1, sl]
            plsc.addupdate_scatter(acc_a, [s >> 7, s & 127], ones)
            plsc.addupdate_scatter(acc_b, [d >> 7, d & 127], ones)
            pk[sl] = s | (d << 16)

    with jax.named_scope("deg_reduce"):
        pltpu.sync_copy(acc_a, sh_a.at[row_idx], add=True)
        pltpu.sync_copy(acc_b, sh_b.at[row_idx], add=True)
        plsc.subcore_barrier()

    rbase = sid * RPT
    pltpu.sync_copy(sh_a.at[pl.ds(rbase, RPT)], invo_s)
    pltpu.sync_copy(sh_b.at[pl.ds(rbase, RPT)], invi_s)
    plsc.subcore_barrier()
    @pl.when(sid < 10)
    def _():
        pltpu.sync_copy(h0_hbm.at[pl.ds(sid * 8, 8)], sh_b.at[pl.ds(sid * 8, 8)])
    zero = jnp.zeros((LANES,), jnp.float32)
    for r in range(RPT):
        for k in range(VPR):
            sl = pl.ds(k * LANES, LANES)
            dg = invo_s[r, sl]
            invo_s[r, sl] = jnp.where(dg > 0.0, _rsqrt_newton(jnp.maximum(dg, 1.0)), zero)
            dg = invi_s[r, sl]
            invi_s[r, sl] = jnp.where(dg > 0.0, _rsqrt_newton(jnp.maximum(dg, 1.0)), zero)

    plsc.subcore_barrier()
    pltpu.sync_copy(sh_b.at[pl.ds(rbase, RPT)], work_s)
    for r in range(RPT):
        for k in range(VPR):
            sl = pl.ds(k * LANES, LANES)
            c_s[pl.ds(r * CH + k * LANES, LANES)] = work_s[r, sl] * invo_s[r, sl]
    with jax.named_scope("bcast_c0"):
        pltpu.sync_copy(c_s, sh_c.at[pl.ds(sid * SLICE, SLICE)])
        plsc.subcore_barrier()
        @pl.when(sid == 0)
        def _():
            pltpu.sync_copy(acc_c, sh_a)

        pltpu.sync_copy(sh_c, cfull)

    accs = [acc_c, acc_a, acc_b]
    for L in range(3):
        acc = accs[L]
        with jax.named_scope(f"edges{L}"):
            @plsc.parallel_loop(0, NLOOP, unroll=8)
            def edge_body(e):
                v = pk[pl.ds(e * LANES, LANES)]
                s = v & 0xFFFF
                d = v >> 16
                cv = plsc.load_gather(cfull, [s])
                plsc.addupdate_scatter(acc, [d >> 7, d & 127], cv)

        with jax.named_scope(f"reduce{L}"):
            cp = pltpu.async_copy(acc, sh_a.at[row_idx], sem0, add=True)
            if L < 2:
                _zero_acc(accs[L + 1])
            cp.wait()
            plsc.subcore_barrier()

        pltpu.sync_copy(sh_a.at[pl.ds(rbase, RPT)], work_s)
        vsum = jnp.zeros((LANES,), jnp.float32)
        wnext = cvec[...][L] if L < 2 else jnp.float32(0.0)
        for r in range(RPT):
            for k in range(VPR):
                sl = pl.ds(k * LANES, LANES)
                h = jnp.maximum(work_s[r, sl] * invi_s[r, sl], 0.0)
                vsum = vsum + h
                if L < 2:
                    c_s[pl.ds(r * CH + k * LANES, LANES)] = h * wnext * invo_s[r, sl]
        sums3[L, pl.ds(0, LANES)] = vsum
        if L < 2:
            with jax.named_scope(f"bcast_c{L + 1}"):
                pltpu.sync_copy(c_s, sh_c.at[pl.ds(sid * SLICE, SLICE)])
                plsc.subcore_barrier()
                @pl.when(sid == 0)
                def _():
                    pltpu.sync_copy(accs[L + 1], sh_a)

                pltpu.sync_copy(sh_c, cfull)

    pltpu.sync_copy(sums3, sh_sums.at[pl.ds(sid * 3, 3)])
    plsc.subcore_barrier()

    @pl.when((sid == 0) & (cid == 0))
    def _():
        pltpu.sync_copy(sh_sums, tmp_all)
        lane = lax.iota(jnp.int32, LANES)
        out_vec = jnp.zeros((LANES,), jnp.float32)
        for L in range(3):
            acc = jnp.zeros((LANES,), jnp.float32)
            for t in range(NT):
                acc = acc + tmp_all[t * 3 + L, pl.ds(0, LANES)]
            s = jnp.sum(acc)
            out_vec = jnp.where(lane == L, s, out_vec)
        stv[...] = out_vec
        pltpu.sync_copy(stv, out_hbm)


def _sc_call(edge_index, h0_flat, consts):
    mesh = plsc.VectorSubcoreMesh(core_axis_name="c", subcore_axis_name="s",
                                  num_cores=2, num_subcores=NT)
    f = pl.kernel(
        _sc_body,
        out_type=jax.ShapeDtypeStruct((LANES,), jnp.float32),
        mesh=mesh,
        compiler_params=pltpu.CompilerParams(needs_layout_passes=False),
        scratch_types=[
            pltpu.VMEM((2, EMAX), jnp.int32),
            pltpu.VMEM((EMAX,), jnp.int32),
            pltpu.VMEM((NPAD,), jnp.float32),
            pltpu.VMEM((NR, CH), jnp.float32),
            pltpu.VMEM((NR, CH), jnp.float32),
            pltpu.VMEM((NR, CH), jnp.float32),
            pltpu.VMEM((NR,), jnp.int32),
            pltpu.VMEM((LANES,), jnp.float32),
            pltpu.VMEM((RPT, CH), jnp.float32),
            pltpu.VMEM((RPT, CH), jnp.float32),
            pltpu.VMEM((RPT, CH), jnp.float32),
            pltpu.VMEM((SLICE,), jnp.float32),
            pltpu.VMEM((3, CH), jnp.float32),
            pltpu.VMEM((NT * 3, CH), jnp.float32),
            pltpu.VMEM((LANES,), jnp.float32),
            pltpu.SemaphoreType.DMA,
            pltpu.VMEM_SHARED((NR, CH), jnp.float32),
            pltpu.VMEM_SHARED((NR, CH), jnp.float32),
            pltpu.VMEM_SHARED((NPAD,), jnp.float32),
            pltpu.VMEM_SHARED((NT * 3, CH), jnp.float32),
        ],
    )
    return f(edge_index, h0_flat, consts)


_TCBLK = 1024


def _tc_body(x_ref, w_ref, h0_ref, colsum_ref):
    i = pl.program_id(0)
    xb = x_ref[...]
    w = w_ref[...]
    rows = jax.lax.broadcasted_iota(jnp.int32, (_TCBLK, 1), 0) + i * _TCBLK
    xb = jnp.where(rows < N, xb, 0.0)
    h0_ref[...] = jnp.sum(xb * w, axis=1, keepdims=True).reshape(_TCBLK // CH, CH)

    @pl.when(i == 0)
    def _():
        colsum_ref[...] = jnp.zeros_like(colsum_ref)

    colsum_ref[...] += jnp.sum(xb, axis=0, keepdims=True)


def _tc_call(x, w_row):
    grid = NPAD // _TCBLK
    return pl.pallas_call(
        _tc_body,
        grid=(grid,),
        in_specs=[
            pl.BlockSpec((_TCBLK, D), lambda i: (i, 0)),
            pl.BlockSpec((1, D), lambda i: (0, 0)),
        ],
        out_specs=[
            pl.BlockSpec((_TCBLK // CH, CH), lambda i: (i, 0)),
            pl.BlockSpec((1, D), lambda i: (0, 0)),
        ],
        out_shape=[
            jax.ShapeDtypeStruct((NR, CH), jnp.float32),
            jax.ShapeDtypeStruct((1, D), jnp.float32),
        ],
    )(x, w_row)


@jax.jit
def kernel(x, edge_index, W1, W2, W3):
    h0, colsum = _tc_call(x, W1.reshape(1, D))
    consts = jnp.zeros((LANES,), jnp.float32)
    consts = consts.at[0].set(W2[0, 0]).at[1].set(W3[0, 0])
    sums = _sc_call(edge_index, h0, consts)
    inv_n = jnp.float32(1.0 / N)
    return jnp.concatenate([colsum * inv_n, (sums[:3] * inv_n)[None, :]], axis=1)


# speedup vs baseline: 1.0344x; 1.0344x over previous
import jax
import jax.numpy as jnp
from jax import lax
from jax.experimental import pallas as pl
from jax.experimental.pallas import tpu as pltpu
from jax.experimental.pallas import tpu_sc as plsc

N = 10000
E = 320000
D = 128

NT = 16
LANES = 16
CH = 128
NR = 80
NPAD = NR * CH
RPT = NR // NT
SLICE = RPT * CH
VPR = CH // LANES

ECHUNK = 19968
EMAX = E - 15 * ECHUNK
NLOOP = EMAX // LANES
PADNODE = NPAD - 1


def _rsqrt_newton(x):
    i = lax.bitcast_convert_type(x, jnp.int32)
    y = lax.bitcast_convert_type(jnp.int32(0x5F3759DF) - (i >> 1), jnp.float32)
    for _ in range(3):
        y = y * (1.5 - 0.5 * x * y * y)
    return y


def _zero_acc(acc):
    z = jnp.zeros((LANES,), jnp.float32)

    def body(r, _):
        for k in range(VPR):
            acc[r, pl.ds(k * LANES, LANES)] = z
        return 0

    lax.fori_loop(0, NR, body, 0)


def _sc_body(edge_hbm, h0_hbm, consts_hbm, out_hbm,
             se2, pk, cfull, acc_a, acc_b, acc_c, row_idx, cvec,
             invo_s, invi_s, work_s, c_s, sums3, tmp_all, stv, sem0,
             sh_a, sh_b, sh_c, sh_sums):
    sid = lax.axis_index("s")
    cid = lax.axis_index("c")

    with jax.named_scope("p0_stage"):
        cp = pltpu.async_copy(edge_hbm.at[:, pl.ds(sid * ECHUNK, ECHUNK)],
                              se2.at[:, pl.ds(0, ECHUNK)], sem0)
        pltpu.sync_copy(consts_hbm, cvec)
        for k in range(NR // LANES):
            row_idx[pl.ds(k * LANES, LANES)] = lax.iota(jnp.int32, LANES) + (k * LANES)
        _zero_acc(acc_a)
        _zero_acc(acc_b)
        _zero_acc(acc_c)
        cp.wait()

        @pl.when(sid < 15)
        def _():
            padv = jnp.full((LANES,), PADNODE, jnp.int32)
            for k in range((EMAX - ECHUNK) // LANES):
                se2[0, pl.ds(ECHUNK + k * LANES, LANES)] = padv
                se2[1, pl.ds(ECHUNK + k * LANES, LANES)] = padv

        @pl.when(sid == 15)
        def _():
            pltpu.sync_copy(edge_hbm.at[:, pl.ds(E - (EMAX - ECHUNK), EMAX - ECHUNK)],
                            se2.at[:, pl.ds(ECHUNK, EMAX - ECHUNK)])

    @pl.when(sid == 0)
    def _():
        pltpu.sync_copy(acc_a, sh_a)
        pltpu.sync_copy(acc_b, sh_b)

    plsc.subcore_barrier()

    ones = jnp.ones((LANES,), jnp.float32)

    with jax.named_scope("deg_loop"):
        @plsc.parallel_loop(0, NLOOP, unroll=8)
        def deg_body(e):
            sl = pl.ds(e * LANES, LANES)
            s = se2[0, sl]
            d = se2[1, sl]
            plsc.addupdate_scatter(acc_a, [s >> 7, s & 127], ones)
            plsc.addupdate_scatter(acc_b, [d >> 7, d & 127], ones)
            pk[sl] = s | (d << 16)

    with jax.named_scope("deg_reduce"):
        pltpu.sync_copy(acc_a, sh_a.at[row_idx], add=True)
        pltpu.sync_copy(acc_b, sh_b.at[row_idx], add=True)
        plsc.subcore_barrier()

    rbase = sid * RPT
    pltpu.sync_copy(sh_a.at[pl.ds(rbase, RPT)], invo_s)
    pltpu.sync_copy(sh_b.at[pl.ds(rbase, RPT)], invi_s)
    plsc.subcore_barrier()
    @pl.when(sid < 10)
    def _():
        pltpu.sync_copy(h0_hbm.at[pl.ds(sid * 8, 8)], sh_b.at[pl.ds(sid * 8, 8)])
    zero = jnp.zeros((LANES,), jnp.float32)
    for r in range(RPT):
        for k in range(VPR):
            sl = pl.ds(k * LANES, LANES)
            dg = invo_s[r, sl]
            invo_s[r, sl] = jnp.where(dg > 0.0, _rsqrt_newton(jnp.maximum(dg, 1.0)), zero)
            dg = invi_s[r, sl]
            invi_s[r, sl] = jnp.where(dg > 0.0, _rsqrt_newton(jnp.maximum(dg, 1.0)), zero)

    plsc.subcore_barrier()
    pltpu.sync_copy(sh_b.at[pl.ds(rbase, RPT)], work_s)
    for r in range(RPT):
        for k in range(VPR):
            sl = pl.ds(k * LANES, LANES)
            c_s[pl.ds(r * CH + k * LANES, LANES)] = work_s[r, sl] * invo_s[r, sl]
    with jax.named_scope("bcast_c0"):
        pltpu.sync_copy(c_s, sh_c.at[pl.ds(sid * SLICE, SLICE)])
        plsc.subcore_barrier()
        @pl.when(sid == 0)
        def _():
            pltpu.sync_copy(acc_c, sh_a)

        pltpu.sync_copy(sh_c, cfull)

    accs = [acc_c, acc_a, acc_b]
    for L in range(3):
        acc = accs[L]
        with jax.named_scope(f"edges{L}"):
            @plsc.parallel_loop(0, NLOOP, unroll=8)
            def edge_body(e):
                v = pk[pl.ds(e * LANES, LANES)]
                s = v & 0xFFFF
                d = v >> 16
                cv = plsc.load_gather(cfull, [s])
                plsc.addupdate_scatter(acc, [d >> 7, d & 127], cv)

        with jax.named_scope(f"reduce{L}"):
            cp = pltpu.async_copy(acc, sh_a.at[row_idx], sem0, add=True)
            if L < 2:
                _zero_acc(accs[L + 1])
            cp.wait()
            plsc.subcore_barrier()

        pltpu.sync_copy(sh_a.at[pl.ds(rbase, RPT)], work_s)
        vsum = jnp.zeros((LANES,), jnp.float32)
        wnext = cvec[...][L] if L < 2 else jnp.float32(0.0)
        for r in range(RPT):
            for k in range(VPR):
                sl = pl.ds(k * LANES, LANES)
                h = jnp.maximum(work_s[r, sl] * invi_s[r, sl], 0.0)
                vsum = vsum + h
                if L < 2:
                    c_s[pl.ds(r * CH + k * LANES, LANES)] = h * wnext * invo_s[r, sl]
        sums3[L, pl.ds(0, LANES)] = vsum
        if L < 2:
            with jax.named_scope(f"bcast_c{L + 1}"):
                pltpu.sync_copy(c_s, sh_c.at[pl.ds(sid * SLICE, SLICE)])
                plsc.subcore_barrier()
                @pl.when(sid == 0)
                def _():
                    pltpu.sync_copy(accs[L + 1], sh_a)

                pltpu.sync_copy(sh_c, cfull)

    pltpu.sync_copy(sums3, sh_sums.at[pl.ds(sid * 3, 3)])
    plsc.subcore_barrier()

    @pl.when((sid == 0) & (cid == 0))
    def _():
        pltpu.sync_copy(sh_sums, tmp_all)
        lane = lax.iota(jnp.int32, LANES)
        out_vec = jnp.zeros((LANES,), jnp.float32)
        for L in range(3):
            acc = jnp.zeros((LANES,), jnp.float32)
            for t in range(NT):
                acc = acc + tmp_all[t * 3 + L, pl.ds(0, LANES)]
            s = jnp.sum(acc)
            out_vec = jnp.where(lane == L, s, out_vec)
        stv[...] = out_vec
        pltpu.sync_copy(stv, out_hbm)


def _sc_call(edge_index, h0_flat, consts):
    mesh = plsc.VectorSubcoreMesh(core_axis_name="c", subcore_axis_name="s",
                                  num_cores=2, num_subcores=NT)
    f = pl.kernel(
        _sc_body,
        out_type=jax.ShapeDtypeStruct((LANES,), jnp.float32),
        mesh=mesh,
        compiler_params=pltpu.CompilerParams(needs_layout_passes=False),
        scratch_types=[
            pltpu.VMEM((2, EMAX), jnp.int32),
            pltpu.VMEM((EMAX,), jnp.int32),
            pltpu.VMEM((NPAD,), jnp.float32),
            pltpu.VMEM((NR, CH), jnp.float32),
            pltpu.VMEM((NR, CH), jnp.float32),
            pltpu.VMEM((NR, CH), jnp.float32),
            pltpu.VMEM((NR,), jnp.int32),
            pltpu.VMEM((LANES,), jnp.float32),
            pltpu.VMEM((RPT, CH), jnp.float32),
            pltpu.VMEM((RPT, CH), jnp.float32),
            pltpu.VMEM((RPT, CH), jnp.float32),
            pltpu.VMEM((SLICE,), jnp.float32),
            pltpu.VMEM((3, CH), jnp.float32),
            pltpu.VMEM((NT * 3, CH), jnp.float32),
            pltpu.VMEM((LANES,), jnp.float32),
            pltpu.SemaphoreType.DMA,
            pltpu.VMEM_SHARED((NR, CH), jnp.float32),
            pltpu.VMEM_SHARED((NR, CH), jnp.float32),
            pltpu.VMEM_SHARED((NPAD,), jnp.float32),
            pltpu.VMEM_SHARED((NT * 3, CH), jnp.float32),
        ],
    )
    return f(edge_index, h0_flat, consts)


_TCBLK = 2048


def _tc_body(x_ref, w_ref, h0_ref, colsum_ref):
    i = pl.program_id(0)
    xb = x_ref[...]
    w = w_ref[...]
    rows = jax.lax.broadcasted_iota(jnp.int32, (_TCBLK, 1), 0) + i * _TCBLK
    xb = jnp.where(rows < N, xb, 0.0)
    h0_ref[...] = jnp.sum(xb * w, axis=1, keepdims=True).reshape(_TCBLK // CH, CH)

    @pl.when(i == 0)
    def _():
        colsum_ref[...] = jnp.zeros_like(colsum_ref)

    colsum_ref[...] += jnp.sum(xb, axis=0, keepdims=True)


def _tc_call(x, w_row):
    grid = NPAD // _TCBLK
    return pl.pallas_call(
        _tc_body,
        grid=(grid,),
        in_specs=[
            pl.BlockSpec((_TCBLK, D), lambda i: (i, 0)),
            pl.BlockSpec((1, D), lambda i: (0, 0)),
        ],
        out_specs=[
            pl.BlockSpec((_TCBLK // CH, CH), lambda i: (i, 0)),
            pl.BlockSpec((1, D), lambda i: (0, 0)),
        ],
        out_shape=[
            jax.ShapeDtypeStruct((NR, CH), jnp.float32),
            jax.ShapeDtypeStruct((1, D), jnp.float32),
        ],
    )(x, w_row)


@jax.jit
def kernel(x, edge_index, W1, W2, W3):
    h0, colsum = _tc_call(x, W1.reshape(1, D))
    consts = jnp.zeros((LANES,), jnp.float32)
    consts = consts.at[0].set(W2[0, 0]).at[1].set(W3[0, 0])
    sums = _sc_call(edge_index, h0, consts)
    inv_n = jnp.float32(1.0 / N)
    return jnp.concatenate([colsum * inv_n, (sums[:3] * inv_n)[None, :]], axis=1)


# speedup vs baseline: 1.0489x; 1.0141x over previous
import jax
import jax.numpy as jnp
from jax import lax
from jax.experimental import pallas as pl
from jax.experimental.pallas import tpu as pltpu
from jax.experimental.pallas import tpu_sc as plsc

N = 10000
E = 320000
D = 128

NT = 16
LANES = 16
CH = 128
NR = 80
NPAD = NR * CH
RPT = NR // NT
SLICE = RPT * CH
VPR = CH // LANES

ECHUNK = 19968
EMAX = E - 15 * ECHUNK
NLOOP = EMAX // LANES
PADNODE = NPAD - 1


def _rsqrt_newton(x):
    i = lax.bitcast_convert_type(x, jnp.int32)
    y = lax.bitcast_convert_type(jnp.int32(0x5F3759DF) - (i >> 1), jnp.float32)
    for _ in range(3):
        y = y * (1.5 - 0.5 * x * y * y)
    return y


def _zero_acc(acc):
    z = jnp.zeros((LANES,), jnp.float32)

    def body(r, _):
        for k in range(VPR):
            acc[r, pl.ds(k * LANES, LANES)] = z
        return 0

    lax.fori_loop(0, NR, body, 0)


def _sc_body(edge_hbm, h0_hbm, consts_hbm, out_hbm,
             se2, pk, cfull, acc_a, acc_b, acc_c, row_idx, cvec,
             invo_s, invi_s, work_s, c_s, sums3, tmp_all, stv, sem0,
             sh_a, sh_b, sh_c, sh_sums):
    sid = lax.axis_index("s")
    cid = lax.axis_index("c")

    with jax.named_scope("p0_stage"):
        cp = pltpu.async_copy(edge_hbm.at[:, pl.ds(sid * ECHUNK, ECHUNK)],
                              se2.at[:, pl.ds(0, ECHUNK)], sem0)
        pltpu.sync_copy(consts_hbm, cvec)
        for k in range(NR // LANES):
            row_idx[pl.ds(k * LANES, LANES)] = lax.iota(jnp.int32, LANES) + (k * LANES)
        _zero_acc(acc_a)
        _zero_acc(acc_b)
        _zero_acc(acc_c)
        cp.wait()

        @pl.when(sid < 15)
        def _():
            padv = jnp.full((LANES,), PADNODE, jnp.int32)
            for k in range((EMAX - ECHUNK) // LANES):
                se2[0, pl.ds(ECHUNK + k * LANES, LANES)] = padv
                se2[1, pl.ds(ECHUNK + k * LANES, LANES)] = padv

        @pl.when(sid == 15)
        def _():
            pltpu.sync_copy(edge_hbm.at[:, pl.ds(E - (EMAX - ECHUNK), EMAX - ECHUNK)],
                            se2.at[:, pl.ds(ECHUNK, EMAX - ECHUNK)])

    @pl.when(sid == 0)
    def _():
        pltpu.sync_copy(acc_a, sh_a)
        pltpu.sync_copy(acc_b, sh_b)

    plsc.subcore_barrier()

    ones = jnp.ones((LANES,), jnp.float32)

    with jax.named_scope("deg_loop"):
        @plsc.parallel_loop(0, NLOOP, unroll=8)
        def deg_body(e):
            sl = pl.ds(e * LANES, LANES)
            s = se2[0, sl]
            d = se2[1, sl]
            plsc.addupdate_scatter(acc_a, [s >> 7, s & 127], ones)
            plsc.addupdate_scatter(acc_b, [d >> 7, d & 127], ones)
            pk[sl] = s | (d << 16)

    with jax.named_scope("deg_reduce"):
        pltpu.sync_copy(acc_a, sh_a.at[row_idx], add=True)
        pltpu.sync_copy(acc_b, sh_b.at[row_idx], add=True)
        plsc.subcore_barrier()

    rbase = sid * RPT
    pltpu.sync_copy(sh_a.at[pl.ds(rbase, RPT)], invo_s)
    pltpu.sync_copy(sh_b.at[pl.ds(rbase, RPT)], invi_s)
    plsc.subcore_barrier()
    @pl.when(sid < 10)
    def _():
        pltpu.sync_copy(h0_hbm.at[pl.ds(sid * 8, 8)], sh_b.at[pl.ds(sid * 8, 8)])
    zero = jnp.zeros((LANES,), jnp.float32)
    for r in range(RPT):
        for k in range(VPR):
            sl = pl.ds(k * LANES, LANES)
            dg = invo_s[r, sl]
            invo_s[r, sl] = jnp.where(dg > 0.0, _rsqrt_newton(jnp.maximum(dg, 1.0)), zero)
            dg = invi_s[r, sl]
            invi_s[r, sl] = jnp.where(dg > 0.0, _rsqrt_newton(jnp.maximum(dg, 1.0)), zero)

    plsc.subcore_barrier()
    pltpu.sync_copy(sh_b.at[pl.ds(rbase, RPT)], work_s)
    for r in range(RPT):
        for k in range(VPR):
            sl = pl.ds(k * LANES, LANES)
            c_s[pl.ds(r * CH + k * LANES, LANES)] = work_s[r, sl] * invo_s[r, sl]
    with jax.named_scope("bcast_c0"):
        pltpu.sync_copy(c_s, sh_c.at[pl.ds(sid * SLICE, SLICE)])
        plsc.subcore_barrier()
        pltpu.sync_copy(sh_c, cfull)

    accs = [acc_c, acc_a, acc_b]
    for L in range(3):
        acc = accs[L]
        with jax.named_scope(f"zero{L}"):
            @pl.when(sid == 0)
            def _():
                pltpu.sync_copy(acc, sh_a)

            plsc.subcore_barrier()

        with jax.named_scope(f"edges{L}"):
            @plsc.parallel_loop(0, NLOOP, unroll=8)
            def edge_body(e):
                v = pk[pl.ds(e * LANES, LANES)]
                s = v & 0xFFFF
                d = v >> 16
                cv = plsc.load_gather(cfull, [s])
                plsc.addupdate_scatter(acc, [d >> 7, d & 127], cv)

        with jax.named_scope(f"reduce{L}"):
            cp = pltpu.async_copy(acc, sh_a.at[row_idx], sem0, add=True)
            if L < 2:
                _zero_acc(accs[L + 1])
            cp.wait()
            plsc.subcore_barrier()

        pltpu.sync_copy(sh_a.at[pl.ds(rbase, RPT)], work_s)
        vsum = jnp.zeros((LANES,), jnp.float32)
        wnext = cvec[...][L] if L < 2 else jnp.float32(0.0)
        for r in range(RPT):
            for k in range(VPR):
                sl = pl.ds(k * LANES, LANES)
                h = jnp.maximum(work_s[r, sl] * invi_s[r, sl], 0.0)
                vsum = vsum + h
                if L < 2:
                    c_s[pl.ds(r * CH + k * LANES, LANES)] = h * wnext * invo_s[r, sl]
        sums3[L, pl.ds(0, LANES)] = vsum
        if L < 2:
            with jax.named_scope(f"bcast_c{L + 1}"):
                pltpu.sync_copy(c_s, sh_c.at[pl.ds(sid * SLICE, SLICE)])
                plsc.subcore_barrier()
                pltpu.sync_copy(sh_c, cfull)

    pltpu.sync_copy(sums3, sh_sums.at[pl.ds(sid * 3, 3)])
    plsc.subcore_barrier()

    @pl.when((sid == 0) & (cid == 0))
    def _():
        pltpu.sync_copy(sh_sums, tmp_all)
        lane = lax.iota(jnp.int32, LANES)
        out_vec = jnp.zeros((LANES,), jnp.float32)
        for L in range(3):
            acc = jnp.zeros((LANES,), jnp.float32)
            for t in range(NT):
                acc = acc + tmp_all[t * 3 + L, pl.ds(0, LANES)]
            s = jnp.sum(acc)
            out_vec = jnp.where(lane == L, s, out_vec)
        stv[...] = out_vec
        pltpu.sync_copy(stv, out_hbm)


def _sc_call(edge_index, h0_flat, consts):
    mesh = plsc.VectorSubcoreMesh(core_axis_name="c", subcore_axis_name="s",
                                  num_cores=2, num_subcores=NT)
    f = pl.kernel(
        _sc_body,
        out_type=jax.ShapeDtypeStruct((LANES,), jnp.float32),
        mesh=mesh,
        compiler_params=pltpu.CompilerParams(needs_layout_passes=False),
        scratch_types=[
            pltpu.VMEM((2, EMAX), jnp.int32),
            pltpu.VMEM((EMAX,), jnp.int32),
            pltpu.VMEM((NPAD,), jnp.float32),
            pltpu.VMEM((NR, CH), jnp.float32),
            pltpu.VMEM((NR, CH), jnp.float32),
            pltpu.VMEM((NR, CH), jnp.float32),
            pltpu.VMEM((NR,), jnp.int32),
            pltpu.VMEM((LANES,), jnp.float32),
            pltpu.VMEM((RPT, CH), jnp.float32),
            pltpu.VMEM((RPT, CH), jnp.float32),
            pltpu.VMEM((RPT, CH), jnp.float32),
            pltpu.VMEM((SLICE,), jnp.float32),
            pltpu.VMEM((3, CH), jnp.float32),
            pltpu.VMEM((NT * 3, CH), jnp.float32),
            pltpu.VMEM((LANES,), jnp.float32),
            pltpu.SemaphoreType.DMA,
            pltpu.VMEM_SHARED((NR, CH), jnp.float32),
            pltpu.VMEM_SHARED((NR, CH), jnp.float32),
            pltpu.VMEM_SHARED((NPAD,), jnp.float32),
            pltpu.VMEM_SHARED((NT * 3, CH), jnp.float32),
        ],
    )
    return f(edge_index, h0_flat, consts)


_TCBLK = 2048


def _tc_body(x_ref, w_ref, h0_ref, colsum_ref):
    i = pl.program_id(0)
    xb = x_ref[...]
    w = w_ref[...]
    rows = jax.lax.broadcasted_iota(jnp.int32, (_TCBLK, 1), 0) + i * _TCBLK
    xb = jnp.where(rows < N, xb, 0.0)
    h0_ref[...] = jnp.sum(xb * w, axis=1, keepdims=True).reshape(_TCBLK // CH, CH)

    @pl.when(i == 0)
    def _():
        colsum_ref[...] = jnp.zeros_like(colsum_ref)

    colsum_ref[...] += jnp.sum(xb, axis=0, keepdims=True)


def _tc_call(x, w_row):
    grid = NPAD // _TCBLK
    return pl.pallas_call(
        _tc_body,
        grid=(grid,),
        in_specs=[
            pl.BlockSpec((_TCBLK, D), lambda i: (i, 0)),
            pl.BlockSpec((1, D), lambda i: (0, 0)),
        ],
        out_specs=[
            pl.BlockSpec((_TCBLK // CH, CH), lambda i: (i, 0)),
            pl.BlockSpec((1, D), lambda i: (0, 0)),
        ],
        out_shape=[
            jax.ShapeDtypeStruct((NR, CH), jnp.float32),
            jax.ShapeDtypeStruct((1, D), jnp.float32),
        ],
    )(x, w_row)


@jax.jit
def kernel(x, edge_index, W1, W2, W3):
    h0, colsum = _tc_call(x, W1.reshape(1, D))
    consts = jnp.zeros((LANES,), jnp.float32)
    consts = consts.at[0].set(W2[0, 0]).at[1].set(W3[0, 0])
    sums = _sc_call(edge_index, h0, consts)
    inv_n = jnp.float32(1.0 / N)
    return jnp.concatenate([colsum * inv_n, (sums[:3] * inv_n)[None, :]], axis=1)
